# baseline (speedup 1.0000x reference)
import jax
import jax.numpy as jnp
from jax import lax
from jax.experimental import pallas as pl
from jax.experimental.pallas import tpu as pltpu

NUM_CLASS = 2
ATTR_NUM = 4096
HID = 2
OUT = 2
TIME_STEP = 3
NODES = ATTR_NUM + NUM_CLASS

BR = 512
NB = (NODES + BR - 1) // BR
NP = NB * BR
ABF_ROWS = 4096

FEAT = (ATTR_NUM + 1) * NUM_CLASS
RI_OUT = ATTR_NUM + 1
BCT = 256
NBT = 8
RP = 4608


def _prop_kernel(cats_ref, gate_ref, adj_ref, p_ref,
                 out_ref, st_s, aoutT_s, abf_s, atail_s):
    i = pl.program_id(0)
    b = i

    @pl.when(i == 0)
    def _init():
        lanes = lax.broadcasted_iota(jnp.int32, (HID, NP), 1)
        hidr = lax.broadcasted_iota(jnp.int32, (HID, NP), 0)
        cnt = cats_ref[0, 0]
        cur = jnp.minimum(cnt, 12)
        h0 = jnp.where((lanes >= NUM_CLASS) & (lanes < NODES) & (hidr == 0),
                       1.0, 0.0).astype(jnp.float32)

        def body(j, acc):
            idx = cats_ref[0, 1 + j]
            vj = (j < cur).astype(jnp.float32)
            return acc + jnp.where((lanes == idx + NUM_CLASS) & (hidr == 1),
                                   vj, 0.0)

        h0 = lax.fori_loop(0, 12, body, h0)
        h0 = h0 * gate_ref[0, 0]
        st_s[2:4, :] = h0
        st_s[0:2, :] = h0
        aoutT_s[...] = jnp.zeros_like(aoutT_s)

    hT = st_s[0:2, 0:NODES]

    def _block(A, edge):
        ainT_b = lax.dot_general(hT, A, (((1,), (1,)), ((), ())),
                                 preferred_element_type=jnp.float32)
        st_s[4:6, pl.ds(b * BR, BR)] = ainT_b
        hbT = st_s[0:2, pl.ds(b * BR, BR)]
        co = lax.dot_general(hbT, A, (((1,), (0,)), ((), ())),
                             preferred_element_type=jnp.float32)
        aoutT_s[0:HID, 0:NODES] += co
        if edge:
            atail_s[...] = A[0:8].astype(jnp.bfloat16)
        else:
            abf_s[pl.ds(b * BR, BR), :] = A.astype(jnp.bfloat16)

    @pl.when(i < NB - 1)
    def _full_block():
        _block(adj_ref[...], False)

    @pl.when(i == NB - 1)
    def _edge_block():
        rows = lax.broadcasted_iota(jnp.int32, (BR, 1), 0) + (NB - 1) * BR
        _block(jnp.where(rows < NODES, adj_ref[...], 0.0), True)

    @pl.when(i >= NB)
    def _resident_step():
        h16T = hT.astype(jnp.bfloat16)
        CH = 1024
        acc = jnp.zeros((HID, NODES), jnp.float32)
        for c in range(4):
            Ac = abf_s[pl.ds(c * CH, CH), :]
            st_s[4:6, pl.ds(c * CH, CH)] = lax.dot_general(
                h16T, Ac, (((1,), (1,)), ((), ())),
                preferred_element_type=jnp.float32)
            hc16 = h16T[:, c * CH:(c + 1) * CH]
            acc = acc + lax.dot_general(hc16, Ac, (((1,), (0,)), ((), ())),
                                        preferred_element_type=jnp.float32)
        At = atail_s[...]
        st_s[4:6, pl.ds(4 * CH, 8)] = lax.dot_general(
            h16T, At, (((1,), (1,)), ((), ())),
            preferred_element_type=jnp.float32)
        ht16 = st_s[0:2, pl.ds(4 * CH, 8)].astype(jnp.bfloat16)
        acc = acc + lax.dot_general(ht16, At, (((1,), (0,)), ((), ())),
                                    preferred_element_type=jnp.float32)
        aoutT_s[0:HID, 0:NODES] = acc

    @pl.when(i >= NB - 1)
    def _update():
        wzT = p_ref[0:4, 0:2].T
        wrT = p_ref[0:4, 2:4].T
        whT = p_ref[0:4, 4:6].T
        woT = p_ref[0:4, 6:8].T
        uzT = p_ref[4:6, 0:2].T
        urT = p_ref[4:6, 2:4].T
        uhT = p_ref[4:6, 4:6].T
        bzT = p_ref[6:7, 0:2].T
        brT = p_ref[6:7, 2:4].T
        bhT = p_ref[6:7, 4:6].T
        boT = p_ref[6:7, 6:8].T
        h = st_s[0:2, :]
        a_inT = st_s[4:6, :]
        a_outT = aoutT_s[0:2, :]
        aT = jnp.concatenate([a_inT, a_outT], axis=0)
        z = jax.nn.sigmoid(jnp.dot(wzT, aT) + jnp.dot(uzT, h) + bzT)
        r = jax.nn.sigmoid(jnp.dot(wrT, aT) + jnp.dot(urT, h) + brT)
        hc = jnp.tanh(jnp.dot(whT, aT) + jnp.dot(uhT, r * h) + bhT)
        h_new = (1.0 - z) * h + z * hc
        lanes = lax.broadcasted_iota(jnp.int32, (HID, NP), 1)
        h_new = jnp.where(lanes < NODES, h_new, 0.0)
        st_s[0:2, :] = h_new
        aoutT_s[...] = jnp.zeros_like(aoutT_s)

        @pl.when(i == NB + 1)
        def _emit():
            hoT = jnp.concatenate([h_new, st_s[2:4, :]], axis=0)
            outT = jnp.tanh(jnp.dot(woT, hoT) + boT)
            out_ref[...] = outT[:, 0:NODES]


def _head_kernel(featT_ref, briT_ref, wriT_a_ref, wriT_b_ref, wriT_c_ref,
                 w1t_ref, b1_ref, w2t_ref, b2_ref, x_ref, frT_s):
    j = pl.program_id(0)
    ft = featT_ref[...]
    fa = jnp.dot(wriT_a_ref[...], ft, preferred_element_type=jnp.float32)
    fb = jnp.dot(wriT_b_ref[...], ft, preferred_element_type=jnp.float32)
    frT_s[pl.ds(2 * j * BCT, BCT), :] = (
        fa + briT_ref[pl.ds(2 * j * BCT, BCT), :])
    frT_s[pl.ds((2 * j + 1) * BCT, BCT), :] = (
        fb + briT_ref[pl.ds((2 * j + 1) * BCT, BCT), :])

    @pl.when(j == 0)
    def _last_row():
        fc = jnp.dot(wriT_c_ref[...], ft, preferred_element_type=jnp.float32)
        frT_s[pl.ds(2 * NBT * BCT, 8), :] = (
            fc + briT_ref[pl.ds(2 * NBT * BCT, 8), :])

    @pl.when(j == NBT - 1)
    def _tail():
        frT = frT_s[0:RI_OUT, :]
        m = jnp.dot(w1t_ref[...], frT,
                    preferred_element_type=jnp.float32)
        relu = jax.nn.relu(m.T + b1_ref[...])
        x_ref[...] = (jnp.dot(w2t_ref[...], relu.T,
                              preferred_element_type=jnp.float32)
                      + b2_ref[...])


def kernel(full_im, categories, card, scene, adj, Wz, Uz, bz, Wr, Ur, br,
           Wh, Uh, bh, Wo, bo, Wri, bri, W1, b1, W2, b2):
    f32 = jnp.float32
    cats = jnp.asarray(categories).astype(jnp.int32)
    gate = (jnp.asarray(card) != 0).astype(f32).reshape(1, 1)

    P = jnp.zeros((8, 8), f32)
    P = P.at[0:4, 0:2].set(Wz).at[0:4, 2:4].set(Wr).at[0:4, 4:6].set(Wh)
    P = P.at[0:4, 6:8].set(Wo)
    P = P.at[4:6, 0:2].set(Uz).at[4:6, 2:4].set(Ur).at[4:6, 4:6].set(Uh)
    P = P.at[6, 0:2].set(bz).at[6, 2:4].set(br).at[6, 4:6].set(bh)
    P = P.at[6, 6:8].set(bo)

    smem = pl.BlockSpec(memory_space=pltpu.SMEM)

    out = pl.pallas_call(
        _prop_kernel,
        grid=(NB + 2,),
        in_specs=[
            smem,
            smem,
            pl.BlockSpec((BR, NODES),
                         lambda i: (jnp.minimum(i, NB - 1), 0)),
            pl.BlockSpec((8, 8), lambda i: (0, 0)),
        ],
        out_specs=pl.BlockSpec((OUT, NODES), lambda i: (0, 0)),
        out_shape=jax.ShapeDtypeStruct((OUT, NODES), f32),
        scratch_shapes=[
            pltpu.VMEM((8, NP), f32),
            pltpu.VMEM((8, NP), f32),
            pltpu.VMEM((ABF_ROWS, NODES), jnp.bfloat16),
            pltpu.VMEM((8, NODES), jnp.bfloat16),
        ],
    )(cats, gate, adj, P)

    clsT = out[:, :NUM_CLASS]
    obj = out[:, NUM_CLASS:].T.reshape(ATTR_NUM * OUT, 1)
    featT = jnp.concatenate(
        [clsT, jnp.broadcast_to(obj, (ATTR_NUM * OUT, NUM_CLASS))], axis=0)

    x = pl.pallas_call(
        _head_kernel,
        grid=(NBT,),
        in_specs=[
            pl.BlockSpec((FEAT, NUM_CLASS), lambda j: (0, 0)),
            pl.BlockSpec((RP, 1), lambda j: (0, 0)),
            pl.BlockSpec((BCT, FEAT), lambda j: (2 * j, 0)),
            pl.BlockSpec((BCT, FEAT), lambda j: (2 * j + 1, 0)),
            pl.BlockSpec((8, FEAT), lambda j: (2 * NBT * BCT // 8, 0)),
            pl.BlockSpec((NUM_CLASS, RI_OUT), lambda j: (0, 0)),
            pl.BlockSpec((1, NUM_CLASS), lambda j: (0, 0)),
            pl.BlockSpec((1, NUM_CLASS), lambda j: (0, 0)),
            pl.BlockSpec((1, 1), lambda j: (0, 0)),
        ],
        out_specs=pl.BlockSpec((1, NUM_CLASS), lambda j: (0, 0)),
        out_shape=jax.ShapeDtypeStruct((1, NUM_CLASS), f32),
        scratch_shapes=[pltpu.VMEM((RP, NUM_CLASS), f32)],
    )(featT,
      jnp.zeros((RP, 1), f32).at[:RI_OUT, 0].set(bri),
      Wri.T, Wri.T, Wri.T, W1.T,
      b1.reshape(1, NUM_CLASS), W2.reshape(1, NUM_CLASS), b2.reshape(1, 1))

    return x


# speedup vs baseline: 1.0015x; 1.0015x over previous
import jax
import jax.numpy as jnp
from jax import lax
from jax.experimental import pallas as pl
from jax.experimental.pallas import tpu as pltpu

NUM_CLASS = 2
ATTR_NUM = 4096
HID = 2
OUT = 2
TIME_STEP = 3
NODES = ATTR_NUM + NUM_CLASS

BR = 512
NB = (NODES + BR - 1) // BR
NP = NB * BR
ABF_ROWS = 4096

FEAT = (ATTR_NUM + 1) * NUM_CLASS
RI_OUT = ATTR_NUM + 1
BCT = 512
NBT = 8
RP = 4608


def _prop_kernel(cats_ref, gate_ref, adj_ref, p_ref,
                 out_ref, st_s, aoutT_s, abf_s, atail_s):
    i = pl.program_id(0)
    b = i

    @pl.when(i == 0)
    def _init():
        lanes = lax.broadcasted_iota(jnp.int32, (HID, NP), 1)
        hidr = lax.broadcasted_iota(jnp.int32, (HID, NP), 0)
        cnt = cats_ref[0, 0]
        cur = jnp.minimum(cnt, 12)
        h0 = jnp.where((lanes >= NUM_CLASS) & (lanes < NODES) & (hidr == 0),
                       1.0, 0.0).astype(jnp.float32)

        def body(j, acc):
            idx = cats_ref[0, 1 + j]
            vj = (j < cur).astype(jnp.float32)
            return acc + jnp.where((lanes == idx + NUM_CLASS) & (hidr == 1),
                                   vj, 0.0)

        h0 = lax.fori_loop(0, 12, body, h0)
        h0 = h0 * gate_ref[0, 0]
        st_s[2:4, :] = h0
        st_s[0:2, :] = h0
        aoutT_s[...] = jnp.zeros_like(aoutT_s)

    hT = st_s[0:2, 0:NODES]

    def _block(A, edge):
        ainT_b = lax.dot_general(hT, A, (((1,), (1,)), ((), ())),
                                 preferred_element_type=jnp.float32)
        st_s[4:6, pl.ds(b * BR, BR)] = ainT_b
        hbT = st_s[0:2, pl.ds(b * BR, BR)]
        co = lax.dot_general(hbT, A, (((1,), (0,)), ((), ())),
                             preferred_element_type=jnp.float32)
        aoutT_s[0:HID, 0:NODES] += co
        if edge:
            atail_s[...] = A[0:8].astype(jnp.bfloat16)
        else:
            abf_s[pl.ds(b * BR, BR), :] = A.astype(jnp.bfloat16)

    @pl.when(i < NB - 1)
    def _full_block():
        _block(adj_ref[...], False)

    @pl.when(i == NB - 1)
    def _edge_block():
        rows = lax.broadcasted_iota(jnp.int32, (BR, 1), 0) + (NB - 1) * BR
        _block(jnp.where(rows < NODES, adj_ref[...], 0.0), True)

    @pl.when(i >= NB)
    def _resident_step():
        h16T = hT.astype(jnp.bfloat16)
        CH = 1024
        acc = jnp.zeros((HID, NODES), jnp.float32)
        for c in range(4):
            Ac = abf_s[pl.ds(c * CH, CH), :]
            st_s[4:6, pl.ds(c * CH, CH)] = lax.dot_general(
                h16T, Ac, (((1,), (1,)), ((), ())),
                preferred_element_type=jnp.float32)
            hc16 = h16T[:, c * CH:(c + 1) * CH]
            acc = acc + lax.dot_general(hc16, Ac, (((1,), (0,)), ((), ())),
                                        preferred_element_type=jnp.float32)
        At = atail_s[...]
        st_s[4:6, pl.ds(4 * CH, 8)] = lax.dot_general(
            h16T, At, (((1,), (1,)), ((), ())),
            preferred_element_type=jnp.float32)
        ht16 = st_s[0:2, pl.ds(4 * CH, 8)].astype(jnp.bfloat16)
        acc = acc + lax.dot_general(ht16, At, (((1,), (0,)), ((), ())),
                                    preferred_element_type=jnp.float32)
        aoutT_s[0:HID, 0:NODES] = acc

    @pl.when(i >= NB - 1)
    def _update():
        wzT = p_ref[0:4, 0:2].T
        wrT = p_ref[0:4, 2:4].T
        whT = p_ref[0:4, 4:6].T
        woT = p_ref[0:4, 6:8].T
        uzT = p_ref[4:6, 0:2].T
        urT = p_ref[4:6, 2:4].T
        uhT = p_ref[4:6, 4:6].T
        bzT = p_ref[6:7, 0:2].T
        brT = p_ref[6:7, 2:4].T
        bhT = p_ref[6:7, 4:6].T
        boT = p_ref[6:7, 6:8].T
        h = st_s[0:2, :]
        a_inT = st_s[4:6, :]
        a_outT = aoutT_s[0:2, :]
        aT = jnp.concatenate([a_inT, a_outT], axis=0)
        z = jax.nn.sigmoid(jnp.dot(wzT, aT) + jnp.dot(uzT, h) + bzT)
        r = jax.nn.sigmoid(jnp.dot(wrT, aT) + jnp.dot(urT, h) + brT)
        hc = jnp.tanh(jnp.dot(whT, aT) + jnp.dot(uhT, r * h) + bhT)
        h_new = (1.0 - z) * h + z * hc
        lanes = lax.broadcasted_iota(jnp.int32, (HID, NP), 1)
        h_new = jnp.where(lanes < NODES, h_new, 0.0)
        st_s[0:2, :] = h_new
        aoutT_s[...] = jnp.zeros_like(aoutT_s)

        @pl.when(i == NB + 1)
        def _emit():
            hoT = jnp.concatenate([h_new, st_s[2:4, :]], axis=0)
            outT = jnp.tanh(jnp.dot(woT, hoT) + boT)
            out_ref[...] = outT[:, 0:NODES]


def _head_kernel(featT_ref, briT_ref, wriT_a_ref, wriT_c_ref,
                 w1t_ref, b1_ref, w2t_ref, b2_ref, x_ref, frT_s):
    j = pl.program_id(0)
    ft = featT_ref[...]
    fa = jnp.dot(wriT_a_ref[...], ft, preferred_element_type=jnp.float32)
    frT_s[pl.ds(j * BCT, BCT), :] = (
        fa + briT_ref[pl.ds(j * BCT, BCT), :])

    @pl.when(j == 0)
    def _last_row():
        fc = jnp.dot(wriT_c_ref[...], ft, preferred_element_type=jnp.float32)
        frT_s[pl.ds(NBT * BCT, 8), :] = (
            fc + briT_ref[pl.ds(NBT * BCT, 8), :])

    @pl.when(j == NBT - 1)
    def _tail():
        frT = frT_s[0:RI_OUT, :]
        m = jnp.dot(w1t_ref[...], frT,
                    preferred_element_type=jnp.float32)
        relu = jax.nn.relu(m.T + b1_ref[...])
        x_ref[...] = (jnp.dot(w2t_ref[...], relu.T,
                              preferred_element_type=jnp.float32)
                      + b2_ref[...])


def kernel(full_im, categories, card, scene, adj, Wz, Uz, bz, Wr, Ur, br,
           Wh, Uh, bh, Wo, bo, Wri, bri, W1, b1, W2, b2):
    f32 = jnp.float32
    cats = jnp.asarray(categories).astype(jnp.int32)
    gate = (jnp.asarray(card) != 0).astype(f32).reshape(1, 1)

    P = jnp.zeros((8, 8), f32)
    P = P.at[0:4, 0:2].set(Wz).at[0:4, 2:4].set(Wr).at[0:4, 4:6].set(Wh)
    P = P.at[0:4, 6:8].set(Wo)
    P = P.at[4:6, 0:2].set(Uz).at[4:6, 2:4].set(Ur).at[4:6, 4:6].set(Uh)
    P = P.at[6, 0:2].set(bz).at[6, 2:4].set(br).at[6, 4:6].set(bh)
    P = P.at[6, 6:8].set(bo)

    smem = pl.BlockSpec(memory_space=pltpu.SMEM)

    out = pl.pallas_call(
        _prop_kernel,
        grid=(NB + 2,),
        in_specs=[
            smem,
            smem,
            pl.BlockSpec((BR, NODES),
                         lambda i: (jnp.minimum(i, NB - 1), 0)),
            pl.BlockSpec((8, 8), lambda i: (0, 0)),
        ],
        out_specs=pl.BlockSpec((OUT, NODES), lambda i: (0, 0)),
        out_shape=jax.ShapeDtypeStruct((OUT, NODES), f32),
        scratch_shapes=[
            pltpu.VMEM((8, NP), f32),
            pltpu.VMEM((8, NP), f32),
            pltpu.VMEM((ABF_ROWS, NODES), jnp.bfloat16),
            pltpu.VMEM((8, NODES), jnp.bfloat16),
        ],
    )(cats, gate, adj, P)

    clsT = out[:, :NUM_CLASS]
    obj = out[:, NUM_CLASS:].T.reshape(ATTR_NUM * OUT, 1)
    featT = jnp.concatenate(
        [clsT, jnp.broadcast_to(obj, (ATTR_NUM * OUT, NUM_CLASS))], axis=0)

    x = pl.pallas_call(
        _head_kernel,
        grid=(NBT,),
        in_specs=[
            pl.BlockSpec((FEAT, NUM_CLASS), lambda j: (0, 0)),
            pl.BlockSpec((RP, 1), lambda j: (0, 0)),
            pl.BlockSpec((BCT, FEAT), lambda j: (j, 0)),
            pl.BlockSpec((8, FEAT), lambda j: (NBT * BCT // 8, 0)),
            pl.BlockSpec((NUM_CLASS, RI_OUT), lambda j: (0, 0)),
            pl.BlockSpec((1, NUM_CLASS), lambda j: (0, 0)),
            pl.BlockSpec((1, NUM_CLASS), lambda j: (0, 0)),
            pl.BlockSpec((1, 1), lambda j: (0, 0)),
        ],
        out_specs=pl.BlockSpec((1, NUM_CLASS), lambda j: (0, 0)),
        out_shape=jax.ShapeDtypeStruct((1, NUM_CLASS), f32),
        scratch_shapes=[pltpu.VMEM((RP, NUM_CLASS), f32)],
    )(featT,
      jnp.zeros((RP, 1), f32).at[:RI_OUT, 0].set(bri),
      Wri.T, Wri.T, W1.T,
      b1.reshape(1, NUM_CLASS), W2.reshape(1, NUM_CLASS), b2.reshape(1, 1))

    return x
